# baseline (speedup 1.0000x reference)
import functools

import jax
import jax.numpy as jnp
from jax import lax
from jax.experimental import pallas as pl
from jax.experimental.pallas import tpu as pltpu
from jax.experimental.pallas import tpu_sc as plsc

D = 32
L = 16
EPS = 1e-8


def _rsqrt(x):
    i = plsc.bitcast(x, jnp.int32)
    i = jnp.int32(0x5F3759DF) - lax.shift_right_arithmetic(i, 1)
    y = plsc.bitcast(i, jnp.float32)
    for _ in range(3):
        y = y * (jnp.float32(1.5) - jnp.float32(0.5) * x * y * y)
    return y


def _make_sc_call(B, NC, NS):
    NW = NC * NS
    bpw = B // NW
    ngroups = bpw // L
    mesh = plsc.VectorSubcoreMesh(core_axis_name="c", subcore_axis_name="s")

    @functools.partial(
        pl.kernel,
        out_type=jax.ShapeDtypeStruct((B,), jnp.float32),
        mesh=mesh,
        compiler_params=pltpu.CompilerParams(
            needs_layout_passes=False, use_tc_tiling_on_sc=False),
        scratch_types=[
            pltpu.VMEM((bpw,), jnp.int32),
            pltpu.VMEM((bpw,), jnp.int32),
            pltpu.VMEM((bpw, D // 2), jnp.int32),
            pltpu.VMEM((bpw, D // 2), jnp.int32),
            pltpu.VMEM((bpw,), jnp.float32),
            pltpu.VMEM((bpw,), jnp.float32),
            pltpu.VMEM((bpw,), jnp.float32),
            pltpu.SemaphoreType.DMA,
            pltpu.SemaphoreType.DMA,
            pltpu.SemaphoreType.DMA,
            pltpu.SemaphoreType.DMA,
        ],
    )
    def sc_call(users_hbm, movies_hbm, uf_hbm, mf_hbm, ub_hbm, mb_hbm,
                out_hbm, idx_u, idx_m, rows_u, rows_m, bias_u, bias_m,
                out_v, sem0, sem1, sem2, sem3):
        wid = lax.axis_index("s") * NC + lax.axis_index("c")
        base = wid * bpw

        pltpu.sync_copy(users_hbm.at[pl.ds(base, bpw)], idx_u)
        pltpu.sync_copy(movies_hbm.at[pl.ds(base, bpw)], idx_m)
        cp0 = pltpu.async_copy(uf_hbm.at[idx_u], rows_u, sem0)
        cp1 = pltpu.async_copy(mf_hbm.at[idx_m], rows_m, sem1)
        cp2 = pltpu.async_copy(ub_hbm.at[idx_u], bias_u, sem2)
        cp3 = pltpu.async_copy(mb_hbm.at[idx_m], bias_m, sem3)
        cp0.wait()
        cp1.wait()
        cp2.wait()
        cp3.wait()

        lane = lax.iota(jnp.int32, 16)

        def group(g, _):
            row0 = g * L
            rows16 = row0 + lane
            dot = jnp.zeros((L,), jnp.float32)
            nnu = jnp.zeros((L,), jnp.float32)
            nnm = jnp.zeros((L,), jnp.float32)
            himask = jnp.int32(-65536)
            for ch in range(D // 2):
                colv = jnp.full((L,), ch, jnp.int32)
                wu = plsc.load_gather(rows_u, [rows16, colv])
                wm = plsc.load_gather(rows_m, [rows16, colv])
                u0 = plsc.bitcast(lax.shift_left(wu, 16), jnp.float32)
                u1 = plsc.bitcast(wu & himask, jnp.float32)
                m0 = plsc.bitcast(lax.shift_left(wm, 16), jnp.float32)
                m1 = plsc.bitcast(wm & himask, jnp.float32)
                dot = dot + u0 * m0 + u1 * m1
                nnu = nnu + u0 * u0 + u1 * u1
                nnm = nnm + m0 * m0 + m1 * m1
            nnu = jnp.maximum(nnu, jnp.float32(1e-24))
            nnm = jnp.maximum(nnm, jnp.float32(1e-24))
            ru = _rsqrt(nnu)
            rm = _rsqrt(nnm)
            nu = nnu * ru
            nm = nnm * rm
            su = jnp.minimum(jnp.float32(1.0), ru)
            sm = jnp.minimum(jnp.float32(1.0), rm)
            denom = jnp.maximum(nu * su * nm * sm, jnp.float32(EPS))
            cos = dot * su * sm / denom
            bu = bias_u[pl.ds(row0, L)]
            bm = bias_m[pl.ds(row0, L)]
            bu = bu * jnp.minimum(
                jnp.float32(1.0),
                jnp.float32(2.0) / jnp.maximum(jnp.abs(bu), jnp.float32(1e-7)))
            bm = bm * jnp.minimum(
                jnp.float32(1.0),
                jnp.float32(2.0) / jnp.maximum(jnp.abs(bm), jnp.float32(1e-7)))
            pred = (cos + bu + bm) * jnp.float32(2.0) + jnp.float32(3.0)
            pred = jnp.clip(pred, jnp.float32(1.0), jnp.float32(5.0))
            out_v[pl.ds(row0, L)] = pred
            return _

        lax.fori_loop(0, ngroups, group, None)
        pltpu.sync_copy(out_v, out_hbm.at[pl.ds(base, bpw)])

    return sc_call


def kernel(users, movies, user_factors, movie_factors, user_biases, movie_biases):
    B = users.shape[0]
    info = plsc.get_sparse_core_info()
    sc_call = _make_sc_call(B, info.num_cores, info.num_subcores)
    return sc_call(
        users.astype(jnp.int32),
        movies.astype(jnp.int32),
        lax.bitcast_convert_type(
            user_factors.astype(jnp.bfloat16).reshape(-1, D // 2, 2),
            jnp.int32),
        lax.bitcast_convert_type(
            movie_factors.astype(jnp.bfloat16).reshape(-1, D // 2, 2),
            jnp.int32),
        user_biases.reshape(-1),
        movie_biases.reshape(-1),
    )


# speedup vs baseline: 2.2382x; 2.2382x over previous
import functools

import jax
import jax.numpy as jnp
from jax import lax
from jax.experimental import pallas as pl
from jax.experimental.pallas import tpu as pltpu
from jax.experimental.pallas import tpu_sc as plsc

D = 32
L = 16
EPS = 1e-8


def _rsqrt(x):
    i = plsc.bitcast(x, jnp.int32)
    i = jnp.int32(0x5F3759DF) - lax.shift_right_arithmetic(i, 1)
    y = plsc.bitcast(i, jnp.float32)
    for _ in range(3):
        y = y * (jnp.float32(1.5) - jnp.float32(0.5) * x * y * y)
    return y


def _make_sc_call(B, NC, NS):
    NW = NC * NS
    bpw = B // NW
    ngroups = bpw // L
    mesh = plsc.VectorSubcoreMesh(core_axis_name="c", subcore_axis_name="s")

    @functools.partial(
        pl.kernel,
        out_type=jax.ShapeDtypeStruct((B,), jnp.float32),
        mesh=mesh,
        compiler_params=pltpu.CompilerParams(
            needs_layout_passes=False, use_tc_tiling_on_sc=False),
        scratch_types=[
            pltpu.VMEM((bpw,), jnp.int32),
            pltpu.VMEM((bpw,), jnp.int32),
            pltpu.VMEM((bpw, D), jnp.float32),
            pltpu.VMEM((bpw, D), jnp.float32),
            pltpu.VMEM((bpw,), jnp.float32),
            pltpu.VMEM((bpw,), jnp.float32),
            pltpu.VMEM((bpw,), jnp.float32),
            pltpu.SemaphoreType.DMA,
            pltpu.SemaphoreType.DMA,
            pltpu.SemaphoreType.DMA,
            pltpu.SemaphoreType.DMA,
        ],
    )
    def sc_call(users_hbm, movies_hbm, uf_hbm, mf_hbm, ub_hbm, mb_hbm,
                out_hbm, idx_u, idx_m, rows_u, rows_m, bias_u, bias_m,
                out_v, sem0, sem1, sem2, sem3):
        wid = lax.axis_index("s") * NC + lax.axis_index("c")
        base = wid * bpw

        pltpu.sync_copy(users_hbm.at[pl.ds(base, bpw)], idx_u)
        pltpu.sync_copy(movies_hbm.at[pl.ds(base, bpw)], idx_m)
        cp0 = pltpu.async_copy(uf_hbm.at[idx_u], rows_u, sem0)
        cp1 = pltpu.async_copy(mf_hbm.at[idx_m], rows_m, sem1)
        cp2 = pltpu.async_copy(ub_hbm.at[idx_u], bias_u, sem2)
        cp3 = pltpu.async_copy(mb_hbm.at[idx_m], bias_m, sem3)
        cp0.wait()
        cp1.wait()
        cp2.wait()
        cp3.wait()

        lane = lax.iota(jnp.int32, 16)

        def group(g, _):
            row0 = g * L
            rows16 = row0 + lane
            dot = jnp.zeros((L,), jnp.float32)
            nnu = jnp.zeros((L,), jnp.float32)
            nnm = jnp.zeros((L,), jnp.float32)
            for c in range(D):
                colv = jnp.full((L,), c, jnp.int32)
                u = plsc.load_gather(rows_u, [rows16, colv])
                m = plsc.load_gather(rows_m, [rows16, colv])
                dot = dot + u * m
                nnu = nnu + u * u
                nnm = nnm + m * m
            nnu = jnp.maximum(nnu, jnp.float32(1e-24))
            nnm = jnp.maximum(nnm, jnp.float32(1e-24))
            ru = _rsqrt(nnu)
            rm = _rsqrt(nnm)
            nu = nnu * ru
            nm = nnm * rm
            su = jnp.minimum(jnp.float32(1.0), ru)
            sm = jnp.minimum(jnp.float32(1.0), rm)
            denom = jnp.maximum(nu * su * nm * sm, jnp.float32(EPS))
            cos = dot * su * sm / denom
            bu = bias_u[pl.ds(row0, L)]
            bm = bias_m[pl.ds(row0, L)]
            bu = bu * jnp.minimum(
                jnp.float32(1.0),
                jnp.float32(2.0) / jnp.maximum(jnp.abs(bu), jnp.float32(1e-7)))
            bm = bm * jnp.minimum(
                jnp.float32(1.0),
                jnp.float32(2.0) / jnp.maximum(jnp.abs(bm), jnp.float32(1e-7)))
            pred = (cos + bu + bm) * jnp.float32(2.0) + jnp.float32(3.0)
            pred = jnp.clip(pred, jnp.float32(1.0), jnp.float32(5.0))
            out_v[pl.ds(row0, L)] = pred
            return _

        lax.fori_loop(0, ngroups, group, None)
        pltpu.sync_copy(out_v, out_hbm.at[pl.ds(base, bpw)])

    return sc_call


def kernel(users, movies, user_factors, movie_factors, user_biases, movie_biases):
    B = users.shape[0]
    info = plsc.get_sparse_core_info()
    sc_call = _make_sc_call(B, info.num_cores, info.num_subcores)
    return sc_call(
        users.astype(jnp.int32),
        movies.astype(jnp.int32),
        user_factors,
        movie_factors,
        user_biases.reshape(-1),
        movie_biases.reshape(-1),
    )
